# Initial kernel scaffold; baseline (speedup 1.0000x reference)
import functools

import jax
import jax.numpy as jnp
from jax import lax
from jax.experimental import pallas as pl
from jax.experimental.pallas import tpu as pltpu
from jax.experimental.pallas import tpu_sc as plsc

N = 10000
E = 320000
NC = 2
NS = 16
NW = NC * NS
EPW = E // NW
CHUNK = 80
NCHUNK = EPW // CHUNK
ROWS_PT = N // NS


def _make_propagate(d):
  mesh = plsc.VectorSubcoreMesh(core_axis_name="c", subcore_axis_name="s")

  @functools.partial(
      pl.kernel,
      out_type=jax.ShapeDtypeStruct((NC, N, d), jnp.float32),
      mesh=mesh,
      scratch_types=[
          pltpu.VMEM((NCHUNK, CHUNK), jnp.int32),
          pltpu.VMEM((NCHUNK, CHUNK), jnp.int32),
          pltpu.VMEM((CHUNK, d), jnp.float32),
          pltpu.VMEM_SHARED((N, d), jnp.float32),
          pltpu.SemaphoreType.DMA,
      ],
  )
  def propagate(g_hbm, src_hbm, dst_hbm, zeros_hbm, out_hbm,
                src_v, dst_v, rows_v, acc, sem):
    cid = lax.axis_index("c")
    sid = lax.axis_index("s")
    wid = cid * NS + sid
    pltpu.sync_copy(src_hbm.at[wid], src_v)
    pltpu.sync_copy(dst_hbm.at[wid], dst_v)
    row0 = sid * ROWS_PT
    pltpu.sync_copy(zeros_hbm.at[pl.ds(row0, ROWS_PT)],
                    acc.at[pl.ds(row0, ROWS_PT)])
    plsc.subcore_barrier()

    @pl.loop(0, NCHUNK)
    def _chunk(j):
      pltpu.async_copy(g_hbm.at[src_v.at[j]], rows_v, sem).wait()
      pltpu.sync_copy(rows_v, acc.at[dst_v.at[j]], add=True)

    plsc.subcore_barrier()
    pltpu.sync_copy(acc.at[pl.ds(row0, ROWS_PT)],
                    out_hbm.at[cid, pl.ds(row0, ROWS_PT)])

  return propagate


_PROPAGATE = {d: _make_propagate(d) for d in (16, 128, 64, 48)}


def _tc_call(body, out_shape, *args):
  return pl.pallas_call(body, out_shape=out_shape)(*args)


def _deg_body(dp0_ref, dp1_ref, dinv_ref):
  deg = dp0_ref[:, 0:1] + dp1_ref[:, 0:1] + 2.0
  dinv_ref[...] = lax.rsqrt(deg)


def _scale_body(x_ref, dinv_ref, g_ref):
  g_ref[...] = x_ref[...] * dinv_ref[...]


def _layer_relu_mm_body(p_ref, g_ref, dinv_ref, w_ref, b_ref, w2_ref, out_ref):
  s = dinv_ref[...] * (p_ref[0] + p_ref[1] + 2.0 * g_ref[...])
  z = jax.nn.relu(
      jnp.dot(s, w_ref[...], preferred_element_type=jnp.float32) + b_ref[...])
  out_ref[...] = dinv_ref[...] * jnp.dot(
      z, w2_ref[...], preferred_element_type=jnp.float32)


def _layer_relu_ew_mm_body(p_ref, g_ref, dinv_ref, b_ref, w2_ref, out_ref):
  z = jax.nn.relu(
      dinv_ref[...] * (p_ref[0] + p_ref[1] + 2.0 * g_ref[...]) + b_ref[...])
  out_ref[...] = dinv_ref[...] * jnp.dot(
      z, w2_ref[...], preferred_element_type=jnp.float32)


def _layer_lin_mm_body(p_ref, g_ref, dinv_ref, b_ref, w2_ref, out_ref):
  z = dinv_ref[...] * (p_ref[0] + p_ref[1] + 2.0 * g_ref[...]) + b_ref[...]
  out_ref[...] = dinv_ref[...] * jnp.dot(
      z, w2_ref[...], preferred_element_type=jnp.float32)


def _final_body(p_ref, g_ref, dinv_ref, b_ref, out_ref):
  out_ref[...] = (dinv_ref[...] * (p_ref[0] + p_ref[1] + 2.0 * g_ref[...])
                  + b_ref[...])


def kernel(x, edge_index, W1, b1, W2, b2, W3, b3, W4, b4):
  src = edge_index[0].reshape(NW, NCHUNK, CHUNK)
  dst = edge_index[1].reshape(NW, NCHUNK, CHUNK)
  zeros128 = jnp.zeros((N, 128), jnp.float32)
  zeros64 = jnp.zeros((N, 64), jnp.float32)
  zeros48 = jnp.zeros((N, 48), jnp.float32)
  zeros16 = jnp.zeros((N, 16), jnp.float32)
  ones16 = jnp.ones((N, 16), jnp.float32)
  W4p = jnp.pad(W4, ((0, 0), (0, 8)))
  b4p = jnp.pad(b4, (0, 8))

  f32 = jnp.float32

  dp = _PROPAGATE[16](ones16, src, dst, zeros16)
  dinv = _tc_call(_deg_body, jax.ShapeDtypeStruct((N, 1), f32), dp[0], dp[1])

  g0 = _tc_call(_scale_body, jax.ShapeDtypeStruct((N, 128), f32), x, dinv)
  s0 = _PROPAGATE[128](g0, src, dst, zeros128)
  g1 = _tc_call(_layer_relu_mm_body, jax.ShapeDtypeStruct((N, 128), f32),
                s0, g0, dinv, W1, b1.reshape(1, -1), W2)

  s1 = _PROPAGATE[128](g1, src, dst, zeros128)
  g2 = _tc_call(_layer_relu_ew_mm_body, jax.ShapeDtypeStruct((N, 64), f32),
                s1, g1, dinv, b2.reshape(1, -1), W3)

  s2 = _PROPAGATE[64](g2, src, dst, zeros64)
  g3 = _tc_call(_layer_lin_mm_body, jax.ShapeDtypeStruct((N, 48), f32),
                s2, g2, dinv, b3.reshape(1, -1), W4p)

  s3 = _PROPAGATE[48](g3, src, dst, zeros48)
  out = _tc_call(_final_body, jax.ShapeDtypeStruct((N, 48), f32),
                 s3, g3, dinv, b4p.reshape(1, -1))
  return out[:, :40]


# speedup vs baseline: 12.5835x; 12.5835x over previous
import functools

import jax
import jax.numpy as jnp
from jax import lax
from jax.experimental import pallas as pl
from jax.experimental.pallas import tpu as pltpu
from jax.experimental.pallas import tpu_sc as plsc

N = 10000
E = 320000
NC = 2
NS = 16
NW = NC * NS
EPW = E // NW
CHUNK = 80
NCHUNK = EPW // CHUNK
ROWS_PT = 624
ROWS_TAIL = N - NS * ROWS_PT


def _make_propagate(d):
  mesh = plsc.VectorSubcoreMesh(core_axis_name="c", subcore_axis_name="s")

  @functools.partial(
      pl.kernel,
      out_type=jax.ShapeDtypeStruct((NC, N, d), jnp.float32),
      mesh=mesh,
      scratch_types=[
          pltpu.VMEM((NCHUNK, CHUNK), jnp.int32),
          pltpu.VMEM((NCHUNK, CHUNK), jnp.int32),
          pltpu.VMEM((CHUNK, d), jnp.float32),
          pltpu.VMEM_SHARED((N, d), jnp.float32),
          pltpu.SemaphoreType.DMA,
      ],
      compiler_params=pltpu.CompilerParams(use_tc_tiling_on_sc=False),
  )
  def propagate(g_hbm, src_hbm, dst_hbm, zeros_hbm, out_hbm,
                src_v, dst_v, rows_v, acc, sem):
    cid = lax.axis_index("c")
    sid = lax.axis_index("s")
    wid = cid * NS + sid
    pltpu.sync_copy(src_hbm.at[wid], src_v)
    pltpu.sync_copy(dst_hbm.at[wid], dst_v)
    row0 = sid * ROWS_PT
    pltpu.sync_copy(zeros_hbm.at[pl.ds(row0, ROWS_PT)],
                    acc.at[pl.ds(row0, ROWS_PT)])

    @pl.when(sid == 0)
    def _zero_tail():
      pltpu.sync_copy(zeros_hbm.at[pl.ds(NS * ROWS_PT, ROWS_TAIL)],
                      acc.at[pl.ds(NS * ROWS_PT, ROWS_TAIL)])

    plsc.subcore_barrier()

    @pl.loop(0, NCHUNK)
    def _chunk(j):
      pltpu.async_copy(g_hbm.at[src_v.at[j]], rows_v, sem).wait()
      pltpu.sync_copy(rows_v, acc.at[dst_v.at[j]], add=True)

    plsc.subcore_barrier()
    pltpu.sync_copy(acc.at[pl.ds(row0, ROWS_PT)],
                    out_hbm.at[cid, pl.ds(row0, ROWS_PT)])

    @pl.when(sid == 0)
    def _copy_tail():
      pltpu.sync_copy(acc.at[pl.ds(NS * ROWS_PT, ROWS_TAIL)],
                      out_hbm.at[cid, pl.ds(NS * ROWS_PT, ROWS_TAIL)])

  return propagate


_PROPAGATE = {d: _make_propagate(d) for d in (16, 128, 64, 48)}


def _tc_call(body, out_shape, *args):
  return pl.pallas_call(body, out_shape=out_shape)(*args)


def _deg_body(dp0_ref, dp1_ref, dinv_ref):
  deg = dp0_ref[:, 0:1] + dp1_ref[:, 0:1] + 2.0
  dinv_ref[...] = lax.rsqrt(deg)


def _scale_body(x_ref, dinv_ref, g_ref):
  g_ref[...] = x_ref[...] * dinv_ref[...]


def _layer_relu_mm_body(p_ref, g_ref, dinv_ref, w_ref, b_ref, w2_ref, out_ref):
  s = dinv_ref[...] * (p_ref[0] + p_ref[1] + 2.0 * g_ref[...])
  z = jax.nn.relu(
      jnp.dot(s, w_ref[...], preferred_element_type=jnp.float32) + b_ref[...])
  out_ref[...] = dinv_ref[...] * jnp.dot(
      z, w2_ref[...], preferred_element_type=jnp.float32)


def _layer_relu_ew_mm_body(p_ref, g_ref, dinv_ref, b_ref, w2_ref, out_ref):
  z = jax.nn.relu(
      dinv_ref[...] * (p_ref[0] + p_ref[1] + 2.0 * g_ref[...]) + b_ref[...])
  out_ref[...] = dinv_ref[...] * jnp.dot(
      z, w2_ref[...], preferred_element_type=jnp.float32)


def _layer_lin_mm_body(p_ref, g_ref, dinv_ref, b_ref, w2_ref, out_ref):
  z = dinv_ref[...] * (p_ref[0] + p_ref[1] + 2.0 * g_ref[...]) + b_ref[...]
  out_ref[...] = dinv_ref[...] * jnp.dot(
      z, w2_ref[...], preferred_element_type=jnp.float32)


def _final_body(p_ref, g_ref, dinv_ref, b_ref, out_ref):
  out_ref[...] = (dinv_ref[...] * (p_ref[0] + p_ref[1] + 2.0 * g_ref[...])
                  + b_ref[...])


def kernel(x, edge_index, W1, b1, W2, b2, W3, b3, W4, b4):
  src = edge_index[0].reshape(NW, NCHUNK, CHUNK)
  dst = edge_index[1].reshape(NW, NCHUNK, CHUNK)
  zeros128 = jnp.zeros((N, 128), jnp.float32)
  zeros64 = jnp.zeros((N, 64), jnp.float32)
  zeros48 = jnp.zeros((N, 48), jnp.float32)
  zeros16 = jnp.zeros((N, 16), jnp.float32)
  ones16 = jnp.ones((N, 16), jnp.float32)
  W4p = jnp.pad(W4, ((0, 0), (0, 8)))
  b4p = jnp.pad(b4, (0, 8))

  f32 = jnp.float32

  dp = _PROPAGATE[16](ones16, src, dst, zeros16)
  dinv = _tc_call(_deg_body, jax.ShapeDtypeStruct((N, 1), f32), dp[0], dp[1])

  g0 = _tc_call(_scale_body, jax.ShapeDtypeStruct((N, 128), f32), x, dinv)
  s0 = _PROPAGATE[128](g0, src, dst, zeros128)
  g1 = _tc_call(_layer_relu_mm_body, jax.ShapeDtypeStruct((N, 128), f32),
                s0, g0, dinv, W1, b1.reshape(1, -1), W2)

  s1 = _PROPAGATE[128](g1, src, dst, zeros128)
  g2 = _tc_call(_layer_relu_ew_mm_body, jax.ShapeDtypeStruct((N, 64), f32),
                s1, g1, dinv, b2.reshape(1, -1), W3)

  s2 = _PROPAGATE[64](g2, src, dst, zeros64)
  g3 = _tc_call(_layer_lin_mm_body, jax.ShapeDtypeStruct((N, 48), f32),
                s2, g2, dinv, b3.reshape(1, -1), W4p)

  s3 = _PROPAGATE[48](g3, src, dst, zeros48)
  out = _tc_call(_final_body, jax.ShapeDtypeStruct((N, 48), f32),
                 s3, g3, dinv, b4p.reshape(1, -1))
  return out[:, :40]


# speedup vs baseline: 21.4468x; 1.7044x over previous
import functools

import jax
import jax.numpy as jnp
from jax import lax
from jax.experimental import pallas as pl
from jax.experimental.pallas import tpu as pltpu
from jax.experimental.pallas import tpu_sc as plsc

N = 10000
E = 320000
NC = 2
NS = 16
NW = NC * NS
EPW = E // NW
CHUNK = 80
NCHUNK = EPW // CHUNK
ROWS_PT = 624
ROWS_TAIL = N - NS * ROWS_PT


def _make_propagate(d):
  mesh = plsc.VectorSubcoreMesh(core_axis_name="c", subcore_axis_name="s")

  @functools.partial(
      pl.kernel,
      out_type=jax.ShapeDtypeStruct((NC, N, d), jnp.float32),
      mesh=mesh,
      scratch_types=[
          pltpu.VMEM((NCHUNK, CHUNK), jnp.int32),
          pltpu.VMEM((NCHUNK, CHUNK), jnp.int32),
          pltpu.VMEM((CHUNK, d), jnp.float32),
          pltpu.VMEM((CHUNK, d), jnp.float32),
          pltpu.VMEM_SHARED((N, d), jnp.float32),
          pltpu.SemaphoreType.DMA,
          pltpu.SemaphoreType.DMA,
      ],
      compiler_params=pltpu.CompilerParams(use_tc_tiling_on_sc=False),
  )
  def propagate(g_hbm, src_hbm, dst_hbm, zeros_hbm, out_hbm,
                src_v, dst_v, rows0, rows1, acc, sem0, sem1):
    cid = lax.axis_index("c")
    sid = lax.axis_index("s")
    wid = cid * NS + sid
    pltpu.sync_copy(src_hbm.at[wid], src_v)
    pltpu.sync_copy(dst_hbm.at[wid], dst_v)
    row0 = sid * ROWS_PT
    pltpu.sync_copy(zeros_hbm.at[pl.ds(row0, ROWS_PT)],
                    acc.at[pl.ds(row0, ROWS_PT)])

    @pl.when(sid == 0)
    def _zero_tail():
      pltpu.sync_copy(zeros_hbm.at[pl.ds(NS * ROWS_PT, ROWS_TAIL)],
                      acc.at[pl.ds(NS * ROWS_PT, ROWS_TAIL)])

    plsc.subcore_barrier()

    pltpu.async_copy(g_hbm.at[src_v.at[0]], rows0, sem0)

    @pl.loop(0, NCHUNK - 1, step=2)
    def _pair(j):
      pltpu.async_copy(g_hbm.at[src_v.at[j + 1]], rows1, sem1)
      pltpu.make_async_copy(g_hbm.at[src_v.at[j]], rows0, sem0).wait()
      pltpu.sync_copy(rows0, acc.at[dst_v.at[j]], add=True)
      pltpu.async_copy(g_hbm.at[src_v.at[j + 2]], rows0, sem0)
      pltpu.make_async_copy(g_hbm.at[src_v.at[j + 1]], rows1, sem1).wait()
      pltpu.sync_copy(rows1, acc.at[dst_v.at[j + 1]], add=True)

    pltpu.make_async_copy(g_hbm.at[src_v.at[NCHUNK - 1]], rows0, sem0).wait()
    pltpu.sync_copy(rows0, acc.at[dst_v.at[NCHUNK - 1]], add=True)

    plsc.subcore_barrier()
    pltpu.sync_copy(acc.at[pl.ds(row0, ROWS_PT)],
                    out_hbm.at[cid, pl.ds(row0, ROWS_PT)])

    @pl.when(sid == 0)
    def _copy_tail():
      pltpu.sync_copy(acc.at[pl.ds(NS * ROWS_PT, ROWS_TAIL)],
                      out_hbm.at[cid, pl.ds(NS * ROWS_PT, ROWS_TAIL)])

  return propagate


_PROPAGATE = {d: _make_propagate(d) for d in (128, 64, 48)}

DEG_D = 16
DEG_FIRE = 5


def _make_degree():
  mesh = plsc.VectorSubcoreMesh(core_axis_name="c", subcore_axis_name="s")

  @functools.partial(
      pl.kernel,
      out_type=jax.ShapeDtypeStruct((NC, N, DEG_D), jnp.float32),
      mesh=mesh,
      scratch_types=[
          pltpu.VMEM((NCHUNK, CHUNK), jnp.int32),
          pltpu.VMEM((CHUNK, DEG_D), jnp.float32),
          pltpu.VMEM_SHARED((N, DEG_D), jnp.float32),
          pltpu.SemaphoreType.DMA,
      ],
      compiler_params=pltpu.CompilerParams(use_tc_tiling_on_sc=False),
  )
  def degree(ones_hbm, dst_hbm, zeros_hbm, out_hbm, dst_v, ones_v, acc, sem):
    cid = lax.axis_index("c")
    sid = lax.axis_index("s")
    wid = cid * NS + sid
    pltpu.sync_copy(dst_hbm.at[wid], dst_v)
    pltpu.sync_copy(ones_hbm, ones_v)
    row0 = sid * ROWS_PT
    pltpu.sync_copy(zeros_hbm.at[pl.ds(row0, ROWS_PT)],
                    acc.at[pl.ds(row0, ROWS_PT)])

    @pl.when(sid == 0)
    def _zero_tail():
      pltpu.sync_copy(zeros_hbm.at[pl.ds(NS * ROWS_PT, ROWS_TAIL)],
                      acc.at[pl.ds(NS * ROWS_PT, ROWS_TAIL)])

    plsc.subcore_barrier()

    @pl.loop(0, NCHUNK, step=DEG_FIRE)
    def _group(j):
      for k in range(DEG_FIRE):
        pltpu.async_copy(ones_v, acc.at[dst_v.at[j + k]], sem, add=True)
      for k in range(DEG_FIRE):
        pltpu.make_async_copy(ones_v, acc.at[dst_v.at[j + k]], sem).wait()

    plsc.subcore_barrier()
    pltpu.sync_copy(acc.at[pl.ds(row0, ROWS_PT)],
                    out_hbm.at[cid, pl.ds(row0, ROWS_PT)])

    @pl.when(sid == 0)
    def _copy_tail():
      pltpu.sync_copy(acc.at[pl.ds(NS * ROWS_PT, ROWS_TAIL)],
                      out_hbm.at[cid, pl.ds(NS * ROWS_PT, ROWS_TAIL)])

  return degree


_DEGREE = _make_degree()


def _tc_call(body, out_shape, *args):
  return pl.pallas_call(body, out_shape=out_shape)(*args)


def _deg_body(dp0_ref, dp1_ref, dinv_ref):
  deg = dp0_ref[:, 0:1] + dp1_ref[:, 0:1] + 2.0
  dinv_ref[...] = lax.rsqrt(deg)


def _scale_body(x_ref, dinv_ref, g_ref):
  g_ref[...] = x_ref[...] * dinv_ref[...]


def _layer_relu_mm_body(p_ref, g_ref, dinv_ref, w_ref, b_ref, w2_ref, out_ref):
  s = dinv_ref[...] * (p_ref[0] + p_ref[1] + 2.0 * g_ref[...])
  z = jax.nn.relu(
      jnp.dot(s, w_ref[...], preferred_element_type=jnp.float32) + b_ref[...])
  out_ref[...] = dinv_ref[...] * jnp.dot(
      z, w2_ref[...], preferred_element_type=jnp.float32)


def _layer_relu_ew_mm_body(p_ref, g_ref, dinv_ref, b_ref, w2_ref, out_ref):
  z = jax.nn.relu(
      dinv_ref[...] * (p_ref[0] + p_ref[1] + 2.0 * g_ref[...]) + b_ref[...])
  out_ref[...] = dinv_ref[...] * jnp.dot(
      z, w2_ref[...], preferred_element_type=jnp.float32)


def _layer_lin_mm_body(p_ref, g_ref, dinv_ref, b_ref, w2_ref, out_ref):
  z = dinv_ref[...] * (p_ref[0] + p_ref[1] + 2.0 * g_ref[...]) + b_ref[...]
  out_ref[...] = dinv_ref[...] * jnp.dot(
      z, w2_ref[...], preferred_element_type=jnp.float32)


def _final_body(p_ref, g_ref, dinv_ref, b_ref, out_ref):
  out_ref[...] = (dinv_ref[...] * (p_ref[0] + p_ref[1] + 2.0 * g_ref[...])
                  + b_ref[...])


def kernel(x, edge_index, W1, b1, W2, b2, W3, b3, W4, b4):
  src = edge_index[0].reshape(NW, NCHUNK, CHUNK)
  dst = edge_index[1].reshape(NW, NCHUNK, CHUNK)
  zeros128 = jnp.zeros((N, 128), jnp.float32)
  zeros64 = jnp.zeros((N, 64), jnp.float32)
  zeros48 = jnp.zeros((N, 48), jnp.float32)
  zeros16 = jnp.zeros((N, DEG_D), jnp.float32)
  ones16 = jnp.ones((CHUNK, DEG_D), jnp.float32)
  W4p = jnp.pad(W4, ((0, 0), (0, 8)))
  b4p = jnp.pad(b4, (0, 8))

  f32 = jnp.float32

  dp = _DEGREE(ones16, dst, zeros16)
  dinv = _tc_call(_deg_body, jax.ShapeDtypeStruct((N, 1), f32), dp[0], dp[1])

  g0 = _tc_call(_scale_body, jax.ShapeDtypeStruct((N, 128), f32), x, dinv)
  s0 = _PROPAGATE[128](g0, src, dst, zeros128)
  g1 = _tc_call(_layer_relu_mm_body, jax.ShapeDtypeStruct((N, 128), f32),
                s0, g0, dinv, W1, b1.reshape(1, -1), W2)

  s1 = _PROPAGATE[128](g1, src, dst, zeros128)
  g2 = _tc_call(_layer_relu_ew_mm_body, jax.ShapeDtypeStruct((N, 64), f32),
                s1, g1, dinv, b2.reshape(1, -1), W3)

  s2 = _PROPAGATE[64](g2, src, dst, zeros64)
  g3 = _tc_call(_layer_lin_mm_body, jax.ShapeDtypeStruct((N, 48), f32),
                s2, g2, dinv, b3.reshape(1, -1), W4p)

  s3 = _PROPAGATE[48](g3, src, dst, zeros48)
  out = _tc_call(_final_body, jax.ShapeDtypeStruct((N, 48), f32),
                 s3, g3, dinv, b4p.reshape(1, -1))
  return out[:, :40]
